# baseline (speedup 1.0000x reference)
import functools

import jax
import jax.numpy as jnp
from jax import lax
from jax.experimental import pallas as pl
from jax.experimental.pallas import tpu as pltpu
from jax.experimental.pallas import tpu_sc as plsc

_LANES = 16
_CHUNK_ROWS = 16
_NBUF = 5
_PREFETCH = 3
_UNROLL = 4


def _make_sc_kernel(batch, seq_len, d_model):
    info = plsc.get_sparse_core_info()
    nc, ns = info.num_cores, info.num_subcores
    nw = nc * ns
    rows_per_w = seq_len // nw
    c = _CHUNK_ROWS
    vecs_per_row = d_model // _LANES
    nchunks = rows_per_w // c
    nslots = nchunks * batch
    nbuf = _NBUF
    mesh = plsc.VectorSubcoreMesh(core_axis_name="c", subcore_axis_name="s")

    buf = lambda: pltpu.VMEM((c, d_model), jnp.float32)
    sem = lambda: pltpu.SemaphoreType.DMA

    @functools.partial(
        pl.kernel,
        mesh=mesh,
        out_type=jax.ShapeDtypeStruct((batch, seq_len, d_model), jnp.float32),
        scratch_types=(
            [buf() for _ in range(nbuf)]
            + [buf(), buf()]
            + [sem() for _ in range(2 * nbuf + 2)]
        ),
    )
    def sc_add(x_hbm, pos_hbm, out_hbm, *scratch):
        xo_bufs = scratch[0:nbuf]
        p_bufs = scratch[nbuf:nbuf + 2]
        sems = scratch[nbuf + 2:]
        sx = sems[0:nbuf]
        so = sems[nbuf:2 * nbuf]
        sp = sems[2 * nbuf:]

        wid = lax.axis_index("s") * nc + lax.axis_index("c")
        s0 = wid * rows_per_w

        def x_cp(k):
            j, b, u = k // batch, k % batch, k % nbuf
            return pltpu.make_async_copy(
                x_hbm.at[b, pl.ds(s0 + j * c, c)], xo_bufs[u], sx[u])

        def o_cp(k):
            j, b, u = k // batch, k % batch, k % nbuf
            return pltpu.make_async_copy(
                xo_bufs[u], out_hbm.at[b, pl.ds(s0 + j * c, c)], so[u])

        def p_cp(j):
            return pltpu.make_async_copy(
                pos_hbm.at[pl.ds(s0 + j * c, c)], p_bufs[j % 2], sp[j % 2])

        p_cp(0).start()
        for k0 in range(_PREFETCH):
            x_cp(k0).start()

        for k in range(nslots):
            j, b = k // batch, k % batch
            if b == 0:
                p_cp(j).wait()
                if j + 1 < nchunks:
                    p_cp(j + 1).start()
            x_cp(k).wait()

            xo_v, p_v = xo_bufs[k % nbuf], p_bufs[j % 2]

            for r in range(c):
                @plsc.parallel_loop(0, vecs_per_row, step=1, unroll=_UNROLL)
                def add_body(i, r=r):
                    sl = pl.ds(i * _LANES, _LANES)
                    plsc.addupdate(xo_v.at[r, sl], p_v[r, sl])

            o_cp(k).start()

            if k + _PREFETCH < nslots:
                if k + _PREFETCH - nbuf >= 0:
                    o_cp(k + _PREFETCH - nbuf).wait()
                x_cp(k + _PREFETCH).start()

        for m in range(max(0, nslots - nbuf), nslots):
            o_cp(m).wait()

    return sc_add


def kernel(x, pos_weight):
    batch, seq_len, d_model = x.shape
    sc = _make_sc_kernel(batch, seq_len, d_model)
    return sc(x, pos_weight[:seq_len])


# speedup vs baseline: 1.2223x; 1.2223x over previous
import functools

import jax
import jax.numpy as jnp
from jax import lax
from jax.experimental import pallas as pl
from jax.experimental.pallas import tpu as pltpu
from jax.experimental.pallas import tpu_sc as plsc

_LANES = 16
_CHUNK_ROWS = 16
_NBUF = 5
_PREFETCH = 3
_UNROLL = 4


def _make_sc_kernel(batch, seq_len, d_model):
    info = plsc.get_sparse_core_info()
    nc, ns = info.num_cores, info.num_subcores
    nw = nc * ns
    rows_per_w = seq_len // nw
    c = _CHUNK_ROWS
    vecs_per_row = d_model // _LANES
    nchunks = rows_per_w // c
    nslots = nchunks * batch
    nbuf = _NBUF
    mesh = plsc.VectorSubcoreMesh(core_axis_name="c", subcore_axis_name="s")

    buf = lambda: pltpu.VMEM((c, d_model), jnp.float32)
    sem = lambda: pltpu.SemaphoreType.DMA

    @functools.partial(
        pl.kernel,
        mesh=mesh,
        out_type=jax.ShapeDtypeStruct((batch, seq_len, d_model), jnp.float32),
        scratch_types=(
            [buf() for _ in range(nbuf)]
            + [buf(), buf()]
            + [sem() for _ in range(2 * nbuf + 2)]
        ),
    )
    def sc_add(x_hbm, pos_hbm, out_hbm, *scratch):
        xo_bufs = scratch[0:nbuf]
        p_bufs = scratch[nbuf:nbuf + 2]
        sems = scratch[nbuf + 2:]
        sx = sems[0:nbuf]
        so = sems[nbuf:2 * nbuf]
        sp = sems[2 * nbuf:]

        wid = lax.axis_index("s") * nc + lax.axis_index("c")
        s0 = wid * rows_per_w

        def x_cp(k):
            j, b, u = k // batch, k % batch, k % nbuf
            return pltpu.make_async_copy(
                x_hbm.at[b, pl.ds(s0 + j * c, c)], xo_bufs[u], sx[u])

        def o_cp(k):
            j, b, u = k // batch, k % batch, k % nbuf
            return pltpu.make_async_copy(
                xo_bufs[u], out_hbm.at[b, pl.ds(s0 + j * c, c)], so[u])

        def p_cp(j):
            return pltpu.make_async_copy(
                pos_hbm.at[pl.ds(s0 + j * c, c)], p_bufs[j % 2], sp[j % 2])

        p_cp(0).start()
        for k0 in range(_PREFETCH):
            x_cp(k0).start()

        for k in range(nslots):
            j, b = k // batch, k % batch
            if b == 0:
                p_cp(j).wait()
                if j + 1 < nchunks:
                    p_cp(j + 1).start()
            x_cp(k).wait()

            xo_v, p_v = xo_bufs[k % nbuf], p_bufs[j % 2]

            @plsc.parallel_loop(0, c * vecs_per_row, step=1, unroll=_UNROLL)
            def add_body(i):
                r = i // vecs_per_row
                col = lax.rem(i, vecs_per_row) * _LANES
                sl = pl.ds(col, _LANES)
                plsc.addupdate(xo_v.at[r, sl], p_v[r, sl])

            o_cp(k).start()

            if k + _PREFETCH < nslots:
                if k + _PREFETCH - nbuf >= 0:
                    o_cp(k + _PREFETCH - nbuf).wait()
                x_cp(k + _PREFETCH).start()

        for m in range(max(0, nslots - nbuf), nslots):
            o_cp(m).wait()

    return sc_add


def kernel(x, pos_weight):
    batch, seq_len, d_model = x.shape
    sc = _make_sc_kernel(batch, seq_len, d_model)
    return sc(x, pos_weight[:seq_len])
